# Initial kernel scaffold; baseline (speedup 1.0000x reference)
import jax
import jax.numpy as jnp
from jax.experimental import pallas as pl


def kernel(x, pos, batch, W1, b1, W2, b2):
    raise NotImplementedError("write your pallas kernel here")


# speedup vs baseline: 12.9320x; 12.9320x over previous
import functools

import numpy as np

import jax
import jax.numpy as jnp
from jax import lax
from jax.experimental import pallas as pl
from jax.experimental.pallas import tpu as pltpu
from jax.experimental.pallas import tpu_sc as plsc

_N = 10000
_NP = 10240
_M = 2500
_MP = 2560
_D = 128
_K = 128
_CAP = 1024
_R2 = np.float32(0.2 * 0.2)


def _fps_body(px_ref, py_ref, pz_ref, sel_ref, dists_ref):
    fidx = (lax.broadcasted_iota(jnp.int32, (80, 128), 0) * 128
            + lax.broadcasted_iota(jnp.int32, (80, 128), 1))
    lane = lax.broadcasted_iota(jnp.int32, (1, 128), 1)
    valid = fidx < _N
    dists_ref[...] = jnp.where(valid, jnp.inf, -jnp.inf).astype(jnp.float32)

    def body(i, carry):
        dists = dists_ref[...]
        mval = jnp.max(dists)
        nxt = jnp.min(jnp.where(dists == mval, fidx, jnp.int32(2 ** 30)))
        sel_ref[i] = nxt
        r = nxt // 128
        c = nxt % 128

        def coord(ref):
            row = ref[pl.ds(r, 1), :]
            return jnp.sum(jnp.where(lane == c, row, jnp.float32(0.0)))

        sx, sy, sz = coord(px_ref), coord(py_ref), coord(pz_ref)
        dx = px_ref[...] - sx
        dy = py_ref[...] - sy
        dz = pz_ref[...] - sz
        d = (dx * dx + dy * dy) + dz * dz
        dists_ref[...] = jnp.minimum(dists, d)
        return carry

    lax.fori_loop(0, _M, body, 0)


def _fps(px2, py2, pz2):
    return pl.pallas_call(
        _fps_body,
        out_shape=jax.ShapeDtypeStruct((_M,), jnp.int32),
        out_specs=pl.BlockSpec(memory_space=pltpu.SMEM),
        scratch_shapes=[pltpu.VMEM((80, 128), jnp.float32)],
    )(px2, py2, pz2)


def _pre_body(x_ref, p_ref, w1x_ref, w1p_ref, b1_ref, a_ref):
    a = jnp.dot(x_ref[...].astype(jnp.bfloat16),
                w1x_ref[...].astype(jnp.bfloat16),
                preferred_element_type=jnp.float32)
    p = p_ref[...]
    a = a + (p[:, 0:1] * w1p_ref[0:1, :]
             + p[:, 1:2] * w1p_ref[1:2, :]
             + p[:, 2:3] * w1p_ref[2:3, :])
    a_ref[...] = a + b1_ref[...]


def _pre(xp, posp, w1x, w1p8, b1r):
    blk = 1024
    return pl.pallas_call(
        _pre_body,
        grid=(_NP // blk,),
        in_specs=[
            pl.BlockSpec((blk, _D), lambda i: (i, 0)),
            pl.BlockSpec((blk, 3), lambda i: (i, 0)),
            pl.BlockSpec((_D, _D), lambda i: (0, 0)),
            pl.BlockSpec((8, _D), lambda i: (0, 0)),
            pl.BlockSpec((1, _D), lambda i: (0, 0)),
        ],
        out_specs=pl.BlockSpec((blk, _D), lambda i: (i, 0)),
        out_shape=jax.ShapeDtypeStruct((_NP, _D), jnp.float32),
    )(xp, posp, w1x, w1p8, b1r)


_BD = 128


def _d2_body(c8_ref, pt_ref, d2_ref):
    c8 = c8_ref[...]
    c2 = ((c8[:, 0:1] * c8[:, 0:1] + c8[:, 1:2] * c8[:, 1:2])
          + c8[:, 2:3] * c8[:, 2:3])
    pt = pt_ref[...]
    p2 = ((pt[0:1, :] * pt[0:1, :] + pt[1:2, :] * pt[1:2, :])
          + pt[2:3, :] * pt[2:3, :])
    mm = jnp.dot(c8.astype(jnp.bfloat16), pt.astype(jnp.bfloat16),
                 preferred_element_type=jnp.float32)
    d2_ref[...] = (c2 + p2) - 2.0 * mm


def _d2(cents8, post):
    return pl.pallas_call(
        _d2_body,
        grid=(_MP // _BD,),
        in_specs=[
            pl.BlockSpec((_BD, 8), lambda i: (i, 0)),
            pl.BlockSpec((8, _NP), lambda i: (0, 0)),
        ],
        out_specs=pl.BlockSpec((_BD, _NP), lambda i: (i, 0)),
        out_shape=jax.ShapeDtypeStruct((_MP, _NP), jnp.float32),
    )(cents8, post)


@functools.cache
def _make_sel():
    info = plsc.get_sparse_core_info()
    nc, ns = info.num_cores, info.num_subcores
    nw = nc * ns
    rpw = _MP // nw
    mesh = plsc.VectorSubcoreMesh(core_axis_name="c", subcore_axis_name="s")

    @functools.partial(
        pl.kernel,
        out_type=jax.ShapeDtypeStruct((_MP * _K, _D), jnp.float32),
        mesh=mesh,
        compiler_params=pltpu.CompilerParams(needs_layout_passes=False),
        scratch_types=[
            pltpu.VMEM((_NP,), jnp.float32),
            pltpu.VMEM((_NP,), jnp.float32),
            pltpu.VMEM((_CAP,), jnp.float32),
            pltpu.VMEM((_CAP,), jnp.int32),
            pltpu.VMEM((_K,), jnp.int32),
            pltpu.VMEM((_K, _D), jnp.float32),
            pltpu.VMEM((_K, _D), jnp.float32),
            pltpu.SemaphoreType.DMA,
            pltpu.SemaphoreType.DMA,
            pltpu.SemaphoreType.DMA,
            pltpu.SemaphoreType.DMA,
            pltpu.SemaphoreType.DMA,
        ],
    )
    def sel(d2_h, a_h, g_h,
            d2v0, d2v1, cdv, civ, nbrv, gbuf0, gbuf1,
            semd0, semd1, semg, semw0, semw1):
        wid = lax.axis_index("s") * nc + lax.axis_index("c")
        base = wid * rpw
        iota16 = lax.iota(jnp.int32, 16)
        zz = jnp.zeros((16,), jnp.int32)
        inf16 = jnp.full((16,), jnp.inf, jnp.float32)
        semd = (semd0, semd1)
        semw = (semw0, semw1)
        d2vs = (d2v0, d2v1)
        gbufs = (gbuf0, gbuf1)

        pltpu.make_async_copy(
            d2_h.at[pl.ds(base * _NP, _NP)], d2v0, semd[0]).start()

        def process(row, b):
            d2b = d2vs[b]
            gb = gbufs[b]
            pltpu.make_async_copy(
                d2_h.at[pl.ds(row * _NP, _NP)], d2b, semd[b]).wait()

            @pl.when(row + 1 < jnp.minimum(_M, base + rpw))
            def _():
                pltpu.make_async_copy(
                    d2_h.at[pl.ds((row + 1) * _NP, _NP)],
                    d2vs[1 - b], semd[1 - b]).start()

            def ci(k, carry2):
                cdv[pl.ds(k * 16, 16)] = inf16
                return carry2

            lax.fori_loop(0, _CAP // 16, ci, 0)

            def ni(k, carry2):
                nbrv[pl.ds(k * 16, 16)] = zz
                return carry2

            lax.fori_loop(0, _K // 16, ni, 0)

            def scan(k, cur):
                dv = d2b[pl.ds(k * 16, 16)]
                msk = dv <= _R2
                pc = plsc.all_reduce_population_count(msk)[0]

                @pl.when(pc > 0)
                def _():
                    pos = cur + plsc.cumsum(
                        jnp.where(msk, 1, 0).astype(jnp.int32)) - 1
                    pos = jnp.minimum(pos, _CAP - 1)
                    plsc.store_scatter(cdv, [pos], dv, mask=msk)
                    plsc.store_scatter(civ, [pos], k * 16 + iota16, mask=msk)

                return cur + pc

            cnum = lax.fori_loop(0, _NP // 16, scan, jnp.int32(0))
            nch = (cnum + 15) // 16

            def bis(_, carry2):
                lo, hi = carry2
                mid = (lo + hi) * jnp.float32(0.5)
                mids = jnp.broadcast_to(mid, (16,))

                def cnt(k, acc):
                    dv = cdv[pl.ds(k * 16, 16)]
                    return acc + jnp.where(dv <= mids, 1, 0).astype(jnp.int32)

                accv = lax.fori_loop(0, nch, cnt, jnp.zeros((16,), jnp.int32))
                tot = jnp.sum(accv)
                feas = tot <= _K
                return (jnp.where(feas, mid, lo), jnp.where(feas, hi, mid))

            lo, hi = lax.fori_loop(0, 36, bis, (jnp.float32(-1.0), _R2))
            tfin = jnp.where(cnum > _K, lo, jnp.float32(1.0))
            tfs = jnp.broadcast_to(tfin, (16,))

            def fc(k, cur):
                dv = cdv[pl.ds(k * 16, 16)]
                iv = civ[pl.ds(k * 16, 16)]
                msk = dv <= tfs
                pc = plsc.all_reduce_population_count(msk)[0]

                @pl.when(pc > 0)
                def _():
                    pos = cur + plsc.cumsum(
                        jnp.where(msk, 1, 0).astype(jnp.int32)) - 1
                    pos = jnp.minimum(pos, _K - 1)
                    plsc.store_scatter(nbrv, [pos], iv, mask=msk)

                return cur + pc

            c2n = lax.fori_loop(0, nch, fc, jnp.int32(0))
            n0 = plsc.load_gather(nbrv, [zz])
            c2s = jnp.broadcast_to(c2n, (16,))

            def pad(k, carry2):
                cu = nbrv[pl.ds(k * 16, 16)]
                pv = k * 16 + iota16
                nbrv[pl.ds(k * 16, 16)] = jnp.where(pv < c2s, cu, n0)
                return carry2

            lax.fori_loop(0, _K // 16, pad, 0)

            pltpu.async_copy(a_h.at[nbrv], gb, semg).wait()
            pltpu.make_async_copy(
                gb, g_h.at[pl.ds(row * _K, _K)], semw[b]).start()

        def pair(r2, carry):
            for b in (0, 1):
                row = base + r2 * 2 + b

                @pl.when(row < _M)
                def _():
                    @pl.when(r2 > 0)
                    def _():
                        pltpu.make_async_copy(
                            gbufs[b], g_h.at[pl.ds(row * _K, _K)],
                            semw[b]).wait()

                    process(row, b)

            return carry

        lax.fori_loop(0, rpw // 2, pair, 0)
        pltpu.make_async_copy(
            gbuf0, g_h.at[pl.ds(base * _K, _K)], semw[0]).wait()
        pltpu.make_async_copy(
            gbuf1, g_h.at[pl.ds(base * _K, _K)], semw[1]).wait()

    return sel


_BC = 16


def _conv_body(g_ref, c_ref, w2_ref, b2_ref, w1p_ref, o_ref):
    c = c_ref[...]
    t = (c[:, 0:1] * w1p_ref[0:1, :]
         + c[:, 1:2] * w1p_ref[1:2, :]
         + c[:, 2:3] * w1p_ref[2:3, :])
    tb = jnp.broadcast_to(t[:, None, :], (_BC, _K, _D)).reshape(_BC * _K, _D)
    h1 = jnp.maximum(g_ref[...] - tb, jnp.float32(0.0))
    h2 = jnp.dot(h1.astype(jnp.bfloat16), w2_ref[...].astype(jnp.bfloat16),
                 preferred_element_type=jnp.float32)
    h2 = jnp.maximum(h2 + b2_ref[...], jnp.float32(0.0))
    for i in range(_BC):
        o_ref[i, :] = jnp.max(h2[i * _K:(i + 1) * _K, :], axis=0)


def _conv(gflat, cents, w2, b2r, w1p8):
    return pl.pallas_call(
        _conv_body,
        grid=(_MP // _BC,),
        in_specs=[
            pl.BlockSpec((_BC * _K, _D), lambda i: (i, 0)),
            pl.BlockSpec((_BC, 3), lambda i: (i, 0)),
            pl.BlockSpec((_D, _D), lambda i: (0, 0)),
            pl.BlockSpec((1, _D), lambda i: (0, 0)),
            pl.BlockSpec((8, _D), lambda i: (0, 0)),
        ],
        out_specs=pl.BlockSpec((_BC, _D), lambda i: (i, 0)),
        out_shape=jax.ShapeDtypeStruct((_MP, _D), jnp.float32),
    )(gflat, cents, w2, b2r, w1p8)


def kernel(x, pos, batch, W1, b1, W2, b2):
    pxp = jnp.pad(pos[:, 0], (0, _NP - _N), constant_values=1e9)
    pyp = jnp.pad(pos[:, 1], (0, _NP - _N), constant_values=1e9)
    pzp = jnp.pad(pos[:, 2], (0, _NP - _N), constant_values=1e9)

    sel_idx = _fps(pxp.reshape(80, 128), pyp.reshape(80, 128),
                   pzp.reshape(80, 128))
    idx = jnp.sort(sel_idx)

    xp = jnp.pad(x, ((0, _NP - _N), (0, 0)))
    posp = jnp.stack([pxp, pyp, pzp], axis=1)
    w1x = W1[:_D]
    w1p8 = jnp.pad(W1[_D:], ((0, 5), (0, 0)))
    b1r = b1.reshape(1, _D)
    a = _pre(xp, posp, w1x, w1p8, b1r)

    idxp = jnp.pad(idx, (0, _MP - _M))
    cents = posp[idxp]
    cents8 = jnp.pad(cents, ((0, 0), (0, 5)))
    post = jnp.pad(jnp.transpose(posp), ((0, 5), (0, 0)))
    d2 = _d2(cents8, post)

    g = _make_sel()(d2.reshape(_MP * _NP), a)

    out = _conv(g, cents, W2, b2.reshape(1, _D), w1p8)

    new_pos = cents[:_M]
    new_batch = batch[idx]
    return out[:_M], new_pos, new_batch


# speedup vs baseline: 17.4965x; 1.3530x over previous
import functools

import numpy as np

import jax
import jax.numpy as jnp
from jax import lax
from jax.experimental import pallas as pl
from jax.experimental.pallas import tpu as pltpu
from jax.experimental.pallas import tpu_sc as plsc

_N = 10000
_NP = 10240
_M = 2500
_MP = 2560
_D = 128
_K = 128
_CAP = 1024
_R2 = np.float32(0.2 * 0.2)


def _fps_body(px_ref, py_ref, pz_ref, sel_ref, dists_ref):
    fidx = (lax.broadcasted_iota(jnp.int32, (80, 128), 0) * 128
            + lax.broadcasted_iota(jnp.int32, (80, 128), 1))
    lane = lax.broadcasted_iota(jnp.int32, (1, 128), 1)
    valid = fidx < _N
    dists_ref[...] = jnp.where(valid, jnp.inf, -jnp.inf).astype(jnp.float32)

    def body(i, carry):
        dists = dists_ref[...]
        mval = jnp.max(dists)
        nxt = jnp.min(jnp.where(dists == mval, fidx, jnp.int32(2 ** 30)))
        sel_ref[i] = nxt
        r = nxt // 128
        c = nxt % 128

        def coord(ref):
            row = ref[pl.ds(r, 1), :]
            return jnp.sum(jnp.where(lane == c, row, jnp.float32(0.0)))

        sx, sy, sz = coord(px_ref), coord(py_ref), coord(pz_ref)
        dx = px_ref[...] - sx
        dy = py_ref[...] - sy
        dz = pz_ref[...] - sz
        d = (dx * dx + dy * dy) + dz * dz
        dists_ref[...] = jnp.minimum(dists, d)
        return carry

    lax.fori_loop(0, _M, body, 0)


def _fps(px2, py2, pz2):
    return pl.pallas_call(
        _fps_body,
        out_shape=jax.ShapeDtypeStruct((_M,), jnp.int32),
        out_specs=pl.BlockSpec(memory_space=pltpu.SMEM),
        scratch_shapes=[pltpu.VMEM((80, 128), jnp.float32)],
    )(px2, py2, pz2)


def _pre_body(x_ref, p_ref, w1x_ref, w1p_ref, b1_ref, a_ref):
    a = jnp.dot(x_ref[...].astype(jnp.bfloat16),
                w1x_ref[...].astype(jnp.bfloat16),
                preferred_element_type=jnp.float32)
    p = p_ref[...]
    a = a + (p[:, 0:1] * w1p_ref[0:1, :]
             + p[:, 1:2] * w1p_ref[1:2, :]
             + p[:, 2:3] * w1p_ref[2:3, :])
    a_ref[...] = a + b1_ref[...]


def _pre(xp, posp, w1x, w1p8, b1r):
    blk = 1024
    return pl.pallas_call(
        _pre_body,
        grid=(_NP // blk,),
        in_specs=[
            pl.BlockSpec((blk, _D), lambda i: (i, 0)),
            pl.BlockSpec((blk, 3), lambda i: (i, 0)),
            pl.BlockSpec((_D, _D), lambda i: (0, 0)),
            pl.BlockSpec((8, _D), lambda i: (0, 0)),
            pl.BlockSpec((1, _D), lambda i: (0, 0)),
        ],
        out_specs=pl.BlockSpec((blk, _D), lambda i: (i, 0)),
        out_shape=jax.ShapeDtypeStruct((_NP, _D), jnp.float32),
    )(xp, posp, w1x, w1p8, b1r)


_BD = 128


def _d2_body(c8_ref, pt_ref, d2_ref):
    c8 = c8_ref[...]
    c2 = ((c8[:, 0:1] * c8[:, 0:1] + c8[:, 1:2] * c8[:, 1:2])
          + c8[:, 2:3] * c8[:, 2:3])
    pt = pt_ref[...]
    p2 = ((pt[0:1, :] * pt[0:1, :] + pt[1:2, :] * pt[1:2, :])
          + pt[2:3, :] * pt[2:3, :])
    mm = jnp.dot(c8.astype(jnp.bfloat16), pt.astype(jnp.bfloat16),
                 preferred_element_type=jnp.float32)
    d2_ref[...] = (c2 + p2) - 2.0 * mm


def _d2(cents8, post):
    return pl.pallas_call(
        _d2_body,
        grid=(_MP // _BD,),
        in_specs=[
            pl.BlockSpec((_BD, 8), lambda i: (i, 0)),
            pl.BlockSpec((8, _NP), lambda i: (0, 0)),
        ],
        out_specs=pl.BlockSpec((_BD, _NP), lambda i: (i, 0)),
        out_shape=jax.ShapeDtypeStruct((_MP, _NP), jnp.float32),
    )(cents8, post)


@functools.cache
def _make_sel():
    info = plsc.get_sparse_core_info()
    nc, ns = info.num_cores, info.num_subcores
    nw = nc * ns
    rpw = _MP // nw
    mesh = plsc.VectorSubcoreMesh(core_axis_name="c", subcore_axis_name="s")

    @functools.partial(
        pl.kernel,
        out_type=jax.ShapeDtypeStruct((_MP * _K, _D), jnp.float32),
        mesh=mesh,
        compiler_params=pltpu.CompilerParams(needs_layout_passes=False),
        scratch_types=[
            pltpu.VMEM((_NP,), jnp.float32),
            pltpu.VMEM((_NP,), jnp.float32),
            pltpu.VMEM((_CAP,), jnp.float32),
            pltpu.VMEM((_CAP,), jnp.int32),
            pltpu.VMEM((_K + 16,), jnp.int32),
            pltpu.VMEM((_K,), jnp.int32),
            pltpu.VMEM((_K, _D), jnp.float32),
            pltpu.VMEM((_K, _D), jnp.float32),
            pltpu.SemaphoreType.DMA,
            pltpu.SemaphoreType.DMA,
            pltpu.SemaphoreType.DMA,
            pltpu.SemaphoreType.DMA,
            pltpu.SemaphoreType.DMA,
        ],
    )
    def sel(d2_h, a_h, g_h,
            d2v0, d2v1, cdv, civ, nbrv, nbr2, gbuf0, gbuf1,
            semd0, semd1, semg, semw0, semw1):
        wid = lax.axis_index("s") * nc + lax.axis_index("c")
        base = wid * rpw
        iota16 = lax.iota(jnp.int32, 16)
        zz = jnp.zeros((16,), jnp.int32)
        inf16 = jnp.full((16,), jnp.inf, jnp.float32)
        semd = (semd0, semd1)
        semw = (semw0, semw1)
        d2vs = (d2v0, d2v1)
        gbufs = (gbuf0, gbuf1)

        pltpu.make_async_copy(
            d2_h.at[pl.ds(base * _NP, _NP)], d2v0, semd[0]).start()

        def process(row, b):
            d2b = d2vs[b]
            gb = gbufs[b]
            pltpu.make_async_copy(
                d2_h.at[pl.ds(row * _NP, _NP)], d2b, semd[b]).wait()

            @pl.when(row + 1 < jnp.minimum(_M, base + rpw))
            def _():
                pltpu.make_async_copy(
                    d2_h.at[pl.ds((row + 1) * _NP, _NP)],
                    d2vs[1 - b], semd[1 - b]).start()

            def scan(k, cur):
                dv = d2b[pl.ds(k * 16, 16)]
                msk = dv <= _R2
                pc = plsc.all_reduce_population_count(msk)[0]
                cs = jnp.minimum(cur, _CAP - 16)
                plsc.store_compressed(cdv.at[pl.ds(cs, 16)], dv, mask=msk)
                plsc.store_compressed(civ.at[pl.ds(cs, 16)],
                                      k * 16 + iota16, mask=msk)
                return cur + pc

            cnum = lax.fori_loop(0, _NP // 16, scan, jnp.int32(0))
            cdv[pl.ds(jnp.minimum(cnum, _CAP - 16), 16)] = inf16
            nch = (cnum + 15) // 16

            def bisect(_):
                def bis(_, carry2):
                    lo, hi = carry2
                    mid = (lo + hi) * jnp.float32(0.5)
                    mids = jnp.broadcast_to(mid, (16,))

                    def cnt(k, acc):
                        dv = cdv[pl.ds(k * 16, 16)]
                        return acc + jnp.where(dv <= mids, 1, 0).astype(
                            jnp.int32)

                    accv = lax.fori_loop(0, nch, cnt,
                                         jnp.zeros((16,), jnp.int32))
                    tot = jnp.sum(accv)
                    feas = tot <= _K
                    return (jnp.where(feas, mid, lo),
                            jnp.where(feas, hi, mid))

                lo, _hi = lax.fori_loop(0, 36, bis,
                                        (jnp.float32(-1.0), _R2))
                return lo

            tfin = lax.cond(cnum > _K, bisect,
                            lambda _: jnp.float32(1.0), 0)
            tfs = jnp.broadcast_to(tfin, (16,))

            def fc(k, cur):
                dv = cdv[pl.ds(k * 16, 16)]
                iv = civ[pl.ds(k * 16, 16)]
                msk = dv <= tfs
                pc = plsc.all_reduce_population_count(msk)[0]
                cs = jnp.minimum(cur, _K)
                plsc.store_compressed(nbrv.at[pl.ds(cs, 16)], iv, mask=msk)
                return cur + pc

            c2n = lax.fori_loop(0, nch, fc, jnp.int32(0))
            n0 = plsc.load_gather(nbrv, [zz])
            c2s = jnp.broadcast_to(c2n, (16,))

            def pad(k, carry2):
                cu = nbrv[pl.ds(k * 16, 16)]
                pv = k * 16 + iota16
                nbr2[pl.ds(k * 16, 16)] = jnp.where(pv < c2s, cu, n0)
                return carry2

            lax.fori_loop(0, _K // 16, pad, 0)

            pltpu.async_copy(a_h.at[nbr2], gb, semg).wait()
            pltpu.make_async_copy(
                gb, g_h.at[pl.ds(row * _K, _K)], semw[b]).start()

        def pair(r2, carry):
            for b in (0, 1):
                row = base + r2 * 2 + b

                @pl.when(row < _M)
                def _():
                    @pl.when(r2 > 0)
                    def _():
                        pltpu.make_async_copy(
                            gbufs[b], g_h.at[pl.ds(row * _K, _K)],
                            semw[b]).wait()

                    process(row, b)

            return carry

        lax.fori_loop(0, rpw // 2, pair, 0)
        pltpu.make_async_copy(
            gbuf0, g_h.at[pl.ds(base * _K, _K)], semw[0]).wait()
        pltpu.make_async_copy(
            gbuf1, g_h.at[pl.ds(base * _K, _K)], semw[1]).wait()

    return sel


_BC = 16


def _conv_body(g_ref, c_ref, w2_ref, b2_ref, w1p_ref, o_ref):
    c = c_ref[...]
    t = (c[:, 0:1] * w1p_ref[0:1, :]
         + c[:, 1:2] * w1p_ref[1:2, :]
         + c[:, 2:3] * w1p_ref[2:3, :])
    tb = jnp.broadcast_to(t[:, None, :], (_BC, _K, _D)).reshape(_BC * _K, _D)
    h1 = jnp.maximum(g_ref[...] - tb, jnp.float32(0.0))
    h2 = jnp.dot(h1.astype(jnp.bfloat16), w2_ref[...].astype(jnp.bfloat16),
                 preferred_element_type=jnp.float32)
    h2 = jnp.maximum(h2 + b2_ref[...], jnp.float32(0.0))
    for i in range(_BC):
        o_ref[i, :] = jnp.max(h2[i * _K:(i + 1) * _K, :], axis=0)


def _conv(gflat, cents, w2, b2r, w1p8):
    return pl.pallas_call(
        _conv_body,
        grid=(_MP // _BC,),
        in_specs=[
            pl.BlockSpec((_BC * _K, _D), lambda i: (i, 0)),
            pl.BlockSpec((_BC, 3), lambda i: (i, 0)),
            pl.BlockSpec((_D, _D), lambda i: (0, 0)),
            pl.BlockSpec((1, _D), lambda i: (0, 0)),
            pl.BlockSpec((8, _D), lambda i: (0, 0)),
        ],
        out_specs=pl.BlockSpec((_BC, _D), lambda i: (i, 0)),
        out_shape=jax.ShapeDtypeStruct((_MP, _D), jnp.float32),
    )(gflat, cents, w2, b2r, w1p8)


def kernel(x, pos, batch, W1, b1, W2, b2):
    pxp = jnp.pad(pos[:, 0], (0, _NP - _N), constant_values=1e9)
    pyp = jnp.pad(pos[:, 1], (0, _NP - _N), constant_values=1e9)
    pzp = jnp.pad(pos[:, 2], (0, _NP - _N), constant_values=1e9)

    sel_idx = _fps(pxp.reshape(80, 128), pyp.reshape(80, 128),
                   pzp.reshape(80, 128))
    idx = jnp.sort(sel_idx)

    xp = jnp.pad(x, ((0, _NP - _N), (0, 0)))
    posp = jnp.stack([pxp, pyp, pzp], axis=1)
    w1x = W1[:_D]
    w1p8 = jnp.pad(W1[_D:], ((0, 5), (0, 0)))
    b1r = b1.reshape(1, _D)
    a = _pre(xp, posp, w1x, w1p8, b1r)

    idxp = jnp.pad(idx, (0, _MP - _M))
    cents = posp[idxp]
    cents8 = jnp.pad(cents, ((0, 0), (0, 5)))
    post = jnp.pad(jnp.transpose(posp), ((0, 5), (0, 0)))
    d2 = _d2(cents8, post)

    g = _make_sel()(d2.reshape(_MP * _NP), a)

    out = _conv(g, cents, W2, b2.reshape(1, _D), w1p8)

    new_pos = cents[:_M]
    new_batch = batch[idx]
    return out[:_M], new_pos, new_batch


# speedup vs baseline: 20.2441x; 1.1570x over previous
import functools

import numpy as np

import jax
import jax.numpy as jnp
from jax import lax
from jax.experimental import pallas as pl
from jax.experimental.pallas import tpu as pltpu
from jax.experimental.pallas import tpu_sc as plsc

_N = 10000
_NP = 10240
_M = 2500
_MP = 2560
_D = 128
_K = 128
_CAP = 1024
_R2 = np.float32(0.2 * 0.2)


def _fps_body(px_ref, py_ref, pz_ref, psm_ref, sel_ref, dists_ref):
    fidx = (lax.broadcasted_iota(jnp.int32, (80, 128), 0) * 128
            + lax.broadcasted_iota(jnp.int32, (80, 128), 1))
    valid = fidx < _N
    dists_ref[...] = jnp.where(valid, jnp.inf, -jnp.inf).astype(jnp.float32)

    def body(i, mval):
        dists = dists_ref[...]
        nxt = jnp.min(jnp.where(dists == mval, fidx, jnp.int32(2 ** 30)))
        sel_ref[i] = nxt
        sx = psm_ref[0, nxt]
        sy = psm_ref[1, nxt]
        sz = psm_ref[2, nxt]
        dx = px_ref[...] - sx
        dy = py_ref[...] - sy
        dz = pz_ref[...] - sz
        d = (dx * dx + dy * dy) + dz * dz
        newd = jnp.minimum(dists, d)
        dists_ref[...] = newd
        return jnp.max(newd)

    lax.fori_loop(0, _M, body, jnp.float32(jnp.inf))


def _fps(px2, py2, pz2, post):
    return pl.pallas_call(
        _fps_body,
        in_specs=[
            pl.BlockSpec((80, 128), lambda: (0, 0)),
            pl.BlockSpec((80, 128), lambda: (0, 0)),
            pl.BlockSpec((80, 128), lambda: (0, 0)),
            pl.BlockSpec(memory_space=pltpu.SMEM),
        ],
        out_shape=jax.ShapeDtypeStruct((_M,), jnp.int32),
        out_specs=pl.BlockSpec(memory_space=pltpu.SMEM),
        scratch_shapes=[pltpu.VMEM((80, 128), jnp.float32)],
    )(px2, py2, pz2, post)


def _pre_body(x_ref, p_ref, w1x_ref, w1p_ref, b1_ref, a_ref):
    a = jnp.dot(x_ref[...].astype(jnp.bfloat16),
                w1x_ref[...].astype(jnp.bfloat16),
                preferred_element_type=jnp.float32)
    p = p_ref[...]
    a = a + (p[:, 0:1] * w1p_ref[0:1, :]
             + p[:, 1:2] * w1p_ref[1:2, :]
             + p[:, 2:3] * w1p_ref[2:3, :])
    a_ref[...] = a + b1_ref[...]


def _pre(xp, posp, w1x, w1p8, b1r):
    blk = 1024
    return pl.pallas_call(
        _pre_body,
        grid=(_NP // blk,),
        in_specs=[
            pl.BlockSpec((blk, _D), lambda i: (i, 0)),
            pl.BlockSpec((blk, 3), lambda i: (i, 0)),
            pl.BlockSpec((_D, _D), lambda i: (0, 0)),
            pl.BlockSpec((8, _D), lambda i: (0, 0)),
            pl.BlockSpec((1, _D), lambda i: (0, 0)),
        ],
        out_specs=pl.BlockSpec((blk, _D), lambda i: (i, 0)),
        out_shape=jax.ShapeDtypeStruct((_NP, _D), jnp.float32),
    )(xp, posp, w1x, w1p8, b1r)


_BD = 128


def _d2_body(c8_ref, pt_ref, d2_ref):
    c8 = c8_ref[...]
    c2 = ((c8[:, 0:1] * c8[:, 0:1] + c8[:, 1:2] * c8[:, 1:2])
          + c8[:, 2:3] * c8[:, 2:3])
    pt = pt_ref[...]
    p2 = ((pt[0:1, :] * pt[0:1, :] + pt[1:2, :] * pt[1:2, :])
          + pt[2:3, :] * pt[2:3, :])
    mm = jnp.dot(c8.astype(jnp.bfloat16), pt.astype(jnp.bfloat16),
                 preferred_element_type=jnp.float32)
    d2_ref[...] = (c2 + p2) - 2.0 * mm


def _d2(cents8, post):
    return pl.pallas_call(
        _d2_body,
        grid=(_MP // _BD,),
        in_specs=[
            pl.BlockSpec((_BD, 8), lambda i: (i, 0)),
            pl.BlockSpec((8, _NP), lambda i: (0, 0)),
        ],
        out_specs=pl.BlockSpec((_BD, _NP), lambda i: (i, 0)),
        out_shape=jax.ShapeDtypeStruct((_MP, _NP), jnp.float32),
    )(cents8, post)


@functools.cache
def _make_sel():
    info = plsc.get_sparse_core_info()
    nc, ns = info.num_cores, info.num_subcores
    nw = nc * ns
    rpw = _MP // nw
    mesh = plsc.VectorSubcoreMesh(core_axis_name="c", subcore_axis_name="s")

    @functools.partial(
        pl.kernel,
        out_type=jax.ShapeDtypeStruct((_MP * _K, _D), jnp.float32),
        mesh=mesh,
        compiler_params=pltpu.CompilerParams(needs_layout_passes=False),
        scratch_types=[
            pltpu.VMEM((_NP,), jnp.float32),
            pltpu.VMEM((_NP,), jnp.float32),
            pltpu.VMEM((_CAP,), jnp.float32),
            pltpu.VMEM((_CAP,), jnp.int32),
            pltpu.VMEM((_K + 16,), jnp.int32),
            pltpu.VMEM((_K,), jnp.int32),
            pltpu.VMEM((_K, _D), jnp.float32),
            pltpu.VMEM((_K, _D), jnp.float32),
            pltpu.SemaphoreType.DMA,
            pltpu.SemaphoreType.DMA,
            pltpu.SemaphoreType.DMA,
            pltpu.SemaphoreType.DMA,
            pltpu.SemaphoreType.DMA,
        ],
    )
    def sel(d2_h, a_h, g_h,
            d2v0, d2v1, cdv, civ, nbrv, nbr2, gbuf0, gbuf1,
            semd0, semd1, semg, semw0, semw1):
        wid = lax.axis_index("s") * nc + lax.axis_index("c")
        base = wid * rpw
        iota16 = lax.iota(jnp.int32, 16)
        zz = jnp.zeros((16,), jnp.int32)
        inf16 = jnp.full((16,), jnp.inf, jnp.float32)
        semd = (semd0, semd1)
        semw = (semw0, semw1)
        d2vs = (d2v0, d2v1)
        gbufs = (gbuf0, gbuf1)

        pltpu.make_async_copy(
            d2_h.at[pl.ds(base * _NP, _NP)], d2v0, semd[0]).start()

        def process(row, b):
            d2b = d2vs[b]
            gb = gbufs[b]
            pltpu.make_async_copy(
                d2_h.at[pl.ds(row * _NP, _NP)], d2b, semd[b]).wait()

            @pl.when(row + 1 < jnp.minimum(_M, base + rpw))
            def _():
                pltpu.make_async_copy(
                    d2_h.at[pl.ds((row + 1) * _NP, _NP)],
                    d2vs[1 - b], semd[1 - b]).start()

            def scan(k, cur):
                dv = d2b[pl.ds(k * 16, 16)]
                msk = dv <= _R2
                pc = plsc.all_reduce_population_count(msk)[0]
                cs = jnp.minimum(cur, _CAP - 16)
                plsc.store_compressed(cdv.at[pl.ds(cs, 16)], dv, mask=msk)
                plsc.store_compressed(civ.at[pl.ds(cs, 16)],
                                      k * 16 + iota16, mask=msk)
                return cur + pc

            cnum = lax.fori_loop(0, _NP // 16, scan, jnp.int32(0))
            cdv[pl.ds(jnp.minimum(cnum, _CAP - 16), 16)] = inf16
            nch = (cnum + 15) // 16

            def bisect(_):
                def bis(_, carry2):
                    lo, hi = carry2
                    mid = (lo + hi) * jnp.float32(0.5)
                    mids = jnp.broadcast_to(mid, (16,))

                    def cnt(k, acc):
                        dv = cdv[pl.ds(k * 16, 16)]
                        return acc + jnp.where(dv <= mids, 1, 0).astype(
                            jnp.int32)

                    accv = lax.fori_loop(0, nch, cnt,
                                         jnp.zeros((16,), jnp.int32))
                    tot = jnp.sum(accv)
                    feas = tot <= _K
                    return (jnp.where(feas, mid, lo),
                            jnp.where(feas, hi, mid))

                lo, _hi = lax.fori_loop(0, 36, bis,
                                        (jnp.float32(-1.0), _R2))
                return lo

            tfin = lax.cond(cnum > _K, bisect,
                            lambda _: jnp.float32(1.0), 0)
            tfs = jnp.broadcast_to(tfin, (16,))

            def fc(k, cur):
                dv = cdv[pl.ds(k * 16, 16)]
                iv = civ[pl.ds(k * 16, 16)]
                msk = dv <= tfs
                pc = plsc.all_reduce_population_count(msk)[0]
                cs = jnp.minimum(cur, _K)
                plsc.store_compressed(nbrv.at[pl.ds(cs, 16)], iv, mask=msk)
                return cur + pc

            c2n = lax.fori_loop(0, nch, fc, jnp.int32(0))
            n0 = plsc.load_gather(nbrv, [zz])
            c2s = jnp.broadcast_to(c2n, (16,))

            def pad(k, carry2):
                cu = nbrv[pl.ds(k * 16, 16)]
                pv = k * 16 + iota16
                nbr2[pl.ds(k * 16, 16)] = jnp.where(pv < c2s, cu, n0)
                return carry2

            lax.fori_loop(0, _K // 16, pad, 0)

            pltpu.async_copy(a_h.at[nbr2], gb, semg).wait()
            pltpu.make_async_copy(
                gb, g_h.at[pl.ds(row * _K, _K)], semw[b]).start()

        def pair(r2, carry):
            for b in (0, 1):
                row = base + r2 * 2 + b

                @pl.when(row < _M)
                def _():
                    @pl.when(r2 > 0)
                    def _():
                        pltpu.make_async_copy(
                            gbufs[b], g_h.at[pl.ds(row * _K, _K)],
                            semw[b]).wait()

                    process(row, b)

            return carry

        lax.fori_loop(0, rpw // 2, pair, 0)
        pltpu.make_async_copy(
            gbuf0, g_h.at[pl.ds(base * _K, _K)], semw[0]).wait()
        pltpu.make_async_copy(
            gbuf1, g_h.at[pl.ds(base * _K, _K)], semw[1]).wait()

    return sel


_BC = 16


def _conv_body(g_ref, c_ref, w2_ref, b2_ref, w1p_ref, o_ref):
    c = c_ref[...]
    t = (c[:, 0:1] * w1p_ref[0:1, :]
         + c[:, 1:2] * w1p_ref[1:2, :]
         + c[:, 2:3] * w1p_ref[2:3, :])
    tb = jnp.broadcast_to(t[:, None, :], (_BC, _K, _D)).reshape(_BC * _K, _D)
    h1 = jnp.maximum(g_ref[...] - tb, jnp.float32(0.0))
    h2 = jnp.dot(h1.astype(jnp.bfloat16), w2_ref[...].astype(jnp.bfloat16),
                 preferred_element_type=jnp.float32)
    h2 = jnp.maximum(h2 + b2_ref[...], jnp.float32(0.0))
    for i in range(_BC):
        o_ref[i, :] = jnp.max(h2[i * _K:(i + 1) * _K, :], axis=0)


def _conv(gflat, cents, w2, b2r, w1p8):
    return pl.pallas_call(
        _conv_body,
        grid=(_MP // _BC,),
        in_specs=[
            pl.BlockSpec((_BC * _K, _D), lambda i: (i, 0)),
            pl.BlockSpec((_BC, 3), lambda i: (i, 0)),
            pl.BlockSpec((_D, _D), lambda i: (0, 0)),
            pl.BlockSpec((1, _D), lambda i: (0, 0)),
            pl.BlockSpec((8, _D), lambda i: (0, 0)),
        ],
        out_specs=pl.BlockSpec((_BC, _D), lambda i: (i, 0)),
        out_shape=jax.ShapeDtypeStruct((_MP, _D), jnp.float32),
    )(gflat, cents, w2, b2r, w1p8)


def kernel(x, pos, batch, W1, b1, W2, b2):
    pxp = jnp.pad(pos[:, 0], (0, _NP - _N), constant_values=1e9)
    pyp = jnp.pad(pos[:, 1], (0, _NP - _N), constant_values=1e9)
    pzp = jnp.pad(pos[:, 2], (0, _NP - _N), constant_values=1e9)

    posp = jnp.stack([pxp, pyp, pzp], axis=1)
    post = jnp.pad(jnp.transpose(posp), ((0, 5), (0, 0)))
    sel_idx = _fps(pxp.reshape(80, 128), pyp.reshape(80, 128),
                   pzp.reshape(80, 128), post)
    idx = jnp.sort(sel_idx)

    xp = jnp.pad(x, ((0, _NP - _N), (0, 0)))
    w1x = W1[:_D]
    w1p8 = jnp.pad(W1[_D:], ((0, 5), (0, 0)))
    b1r = b1.reshape(1, _D)
    a = _pre(xp, posp, w1x, w1p8, b1r)

    idxp = jnp.pad(idx, (0, _MP - _M))
    cents = posp[idxp]
    cents8 = jnp.pad(cents, ((0, 0), (0, 5)))
    d2 = _d2(cents8, post)

    g = _make_sel()(d2.reshape(_MP * _NP), a)

    out = _conv(g, cents, W2, b2.reshape(1, _D), w1p8)

    new_pos = cents[:_M]
    new_batch = batch[idx]
    return out[:_M], new_pos, new_batch


# speedup vs baseline: 23.8571x; 1.1785x over previous
import functools

import numpy as np

import jax
import jax.numpy as jnp
from jax import lax
from jax.experimental import pallas as pl
from jax.experimental.pallas import tpu as pltpu
from jax.experimental.pallas import tpu_sc as plsc

_N = 10000
_NP = 10240
_M = 2500
_MP = 2560
_D = 128
_K = 128
_CAP = 1024
_R2 = np.float32(0.2 * 0.2)


def _fps_body(px_ref, py_ref, pz_ref, psm_ref, sel_ref, dists_ref):
    fidx = (lax.broadcasted_iota(jnp.int32, (80, 128), 0) * 128
            + lax.broadcasted_iota(jnp.int32, (80, 128), 1))
    valid = fidx < _N
    dists_ref[...] = jnp.where(valid, jnp.inf, -jnp.inf).astype(jnp.float32)

    def body(i, mval):
        dists = dists_ref[...]
        nxt = jnp.min(jnp.where(dists == mval, fidx, jnp.int32(2 ** 30)))
        sel_ref[i] = nxt
        sx = psm_ref[0, nxt]
        sy = psm_ref[1, nxt]
        sz = psm_ref[2, nxt]
        dx = px_ref[...] - sx
        dy = py_ref[...] - sy
        dz = pz_ref[...] - sz
        d = (dx * dx + dy * dy) + dz * dz
        newd = jnp.minimum(dists, d)
        dists_ref[...] = newd
        return jnp.max(newd)

    lax.fori_loop(0, _M, body, jnp.float32(jnp.inf))


def _fps(px2, py2, pz2, post):
    return pl.pallas_call(
        _fps_body,
        in_specs=[
            pl.BlockSpec((80, 128), lambda: (0, 0)),
            pl.BlockSpec((80, 128), lambda: (0, 0)),
            pl.BlockSpec((80, 128), lambda: (0, 0)),
            pl.BlockSpec(memory_space=pltpu.SMEM),
        ],
        out_shape=jax.ShapeDtypeStruct((_M,), jnp.int32),
        out_specs=pl.BlockSpec(memory_space=pltpu.SMEM),
        scratch_shapes=[pltpu.VMEM((80, 128), jnp.float32)],
    )(px2, py2, pz2, post)


def _pre_body(x_ref, p_ref, w1x_ref, w1p_ref, b1_ref, a_ref):
    a = jnp.dot(x_ref[...].astype(jnp.bfloat16),
                w1x_ref[...].astype(jnp.bfloat16),
                preferred_element_type=jnp.float32)
    p = p_ref[...]
    a = a + (p[:, 0:1] * w1p_ref[0:1, :]
             + p[:, 1:2] * w1p_ref[1:2, :]
             + p[:, 2:3] * w1p_ref[2:3, :])
    a_ref[...] = a + b1_ref[...]


def _pre(xp, posp, w1x, w1p8, b1r):
    blk = 1024
    return pl.pallas_call(
        _pre_body,
        grid=(_NP // blk,),
        in_specs=[
            pl.BlockSpec((blk, _D), lambda i: (i, 0)),
            pl.BlockSpec((blk, 3), lambda i: (i, 0)),
            pl.BlockSpec((_D, _D), lambda i: (0, 0)),
            pl.BlockSpec((8, _D), lambda i: (0, 0)),
            pl.BlockSpec((1, _D), lambda i: (0, 0)),
        ],
        out_specs=pl.BlockSpec((blk, _D), lambda i: (i, 0)),
        out_shape=jax.ShapeDtypeStruct((_NP, _D), jnp.float32),
    )(xp, posp, w1x, w1p8, b1r)


_BD = 128


def _d2_body(c8_ref, pt_ref, d2_ref):
    c8 = c8_ref[...]
    c2 = ((c8[:, 0:1] * c8[:, 0:1] + c8[:, 1:2] * c8[:, 1:2])
          + c8[:, 2:3] * c8[:, 2:3])
    pt = pt_ref[...]
    p2 = ((pt[0:1, :] * pt[0:1, :] + pt[1:2, :] * pt[1:2, :])
          + pt[2:3, :] * pt[2:3, :])
    mm = jnp.dot(c8.astype(jnp.bfloat16), pt.astype(jnp.bfloat16),
                 preferred_element_type=jnp.float32)
    d2_ref[...] = (c2 + p2) - 2.0 * mm


def _d2(cents8, post):
    return pl.pallas_call(
        _d2_body,
        grid=(_MP // _BD,),
        in_specs=[
            pl.BlockSpec((_BD, 8), lambda i: (i, 0)),
            pl.BlockSpec((8, _NP), lambda i: (0, 0)),
        ],
        out_specs=pl.BlockSpec((_BD, _NP), lambda i: (i, 0)),
        out_shape=jax.ShapeDtypeStruct((_MP, _NP), jnp.float32),
    )(cents8, post)


@functools.cache
def _make_sel():
    info = plsc.get_sparse_core_info()
    nc, ns = info.num_cores, info.num_subcores
    nw = nc * ns
    rpw = _MP // nw
    mesh = plsc.VectorSubcoreMesh(core_axis_name="c", subcore_axis_name="s")

    @functools.partial(
        pl.kernel,
        out_type=jax.ShapeDtypeStruct((_MP * _K, _D), jnp.float32),
        mesh=mesh,
        compiler_params=pltpu.CompilerParams(needs_layout_passes=False),
        scratch_types=[
            pltpu.VMEM((_NP,), jnp.float32),
            pltpu.VMEM((_NP,), jnp.float32),
            pltpu.VMEM((_CAP,), jnp.float32),
            pltpu.VMEM((_CAP,), jnp.int32),
            pltpu.VMEM((_K + 16,), jnp.int32),
            pltpu.VMEM((_K,), jnp.int32),
            pltpu.VMEM((_K, _D), jnp.float32),
            pltpu.VMEM((_K, _D), jnp.float32),
            pltpu.SemaphoreType.DMA,
            pltpu.SemaphoreType.DMA,
            pltpu.SemaphoreType.DMA,
            pltpu.SemaphoreType.DMA,
            pltpu.SemaphoreType.DMA,
        ],
    )
    def sel(d2_h, a_h, g_h,
            d2v0, d2v1, cdv, civ, nbrv, nbr2, gbuf0, gbuf1,
            semd0, semd1, semg, semw0, semw1):
        wid = lax.axis_index("s") * nc + lax.axis_index("c")
        base = wid * rpw
        iota16 = lax.iota(jnp.int32, 16)
        zz = jnp.zeros((16,), jnp.int32)
        inf16 = jnp.full((16,), jnp.inf, jnp.float32)
        semd = (semd0, semd1)
        semw = (semw0, semw1)
        d2vs = (d2v0, d2v1)
        gbufs = (gbuf0, gbuf1)

        pltpu.make_async_copy(
            d2_h.at[pl.ds(base * _NP, _NP)], d2v0, semd[0]).start()

        def process(row, b):
            d2b = d2vs[b]
            gb = gbufs[b]
            pltpu.make_async_copy(
                d2_h.at[pl.ds(row * _NP, _NP)], d2b, semd[b]).wait()

            @pl.when(row + 1 < jnp.minimum(_M, base + rpw))
            def _():
                pltpu.make_async_copy(
                    d2_h.at[pl.ds((row + 1) * _NP, _NP)],
                    d2vs[1 - b], semd[1 - b]).start()

            @plsc.parallel_loop(0, _NP // 16, unroll=8, carry=jnp.int32(0))
            def cnum(k, cur):
                dv = d2b[pl.ds(k * 16, 16)]
                msk = dv <= _R2
                pc = plsc.all_reduce_population_count(msk)[0]
                cs = jnp.minimum(cur, _CAP - 16)
                plsc.store_compressed(cdv.at[pl.ds(cs, 16)], dv, mask=msk)
                plsc.store_compressed(civ.at[pl.ds(cs, 16)],
                                      k * 16 + iota16, mask=msk)
                return cur + pc
            cdv[pl.ds(jnp.minimum(cnum, _CAP - 16), 16)] = inf16
            nch = (cnum + 15) // 16

            def bisect(_):
                def bis(_, carry2):
                    lo, hi = carry2
                    mid = (lo + hi) * jnp.float32(0.5)
                    mids = jnp.broadcast_to(mid, (16,))

                    def cnt(k, acc):
                        dv = cdv[pl.ds(k * 16, 16)]
                        return acc + jnp.where(dv <= mids, 1, 0).astype(
                            jnp.int32)

                    accv = lax.fori_loop(0, nch, cnt,
                                         jnp.zeros((16,), jnp.int32))
                    tot = jnp.sum(accv)
                    feas = tot <= _K
                    return (jnp.where(feas, mid, lo),
                            jnp.where(feas, hi, mid))

                lo, _hi = lax.fori_loop(0, 36, bis,
                                        (jnp.float32(-1.0), _R2))
                return lo

            tfin = lax.cond(cnum > _K, bisect,
                            lambda _: jnp.float32(1.0), 0)
            tfs = jnp.broadcast_to(tfin, (16,))

            def fc(k, cur):
                dv = cdv[pl.ds(k * 16, 16)]
                iv = civ[pl.ds(k * 16, 16)]
                msk = dv <= tfs
                pc = plsc.all_reduce_population_count(msk)[0]
                cs = jnp.minimum(cur, _K)
                plsc.store_compressed(nbrv.at[pl.ds(cs, 16)], iv, mask=msk)
                return cur + pc

            c2n = lax.fori_loop(0, nch, fc, jnp.int32(0))
            n0 = plsc.load_gather(nbrv, [zz])
            c2s = jnp.broadcast_to(c2n, (16,))

            def pad(k, carry2):
                cu = nbrv[pl.ds(k * 16, 16)]
                pv = k * 16 + iota16
                nbr2[pl.ds(k * 16, 16)] = jnp.where(pv < c2s, cu, n0)
                return carry2

            lax.fori_loop(0, _K // 16, pad, 0)

            pltpu.async_copy(a_h.at[nbr2], gb, semg).wait()
            pltpu.make_async_copy(
                gb, g_h.at[pl.ds(row * _K, _K)], semw[b]).start()

        def pair(r2, carry):
            for b in (0, 1):
                row = base + r2 * 2 + b

                @pl.when(row < _M)
                def _():
                    @pl.when(r2 > 0)
                    def _():
                        pltpu.make_async_copy(
                            gbufs[b], g_h.at[pl.ds(row * _K, _K)],
                            semw[b]).wait()

                    process(row, b)

            return carry

        lax.fori_loop(0, rpw // 2, pair, 0)
        pltpu.make_async_copy(
            gbuf0, g_h.at[pl.ds(base * _K, _K)], semw[0]).wait()
        pltpu.make_async_copy(
            gbuf1, g_h.at[pl.ds(base * _K, _K)], semw[1]).wait()

    return sel


_BC = 16


def _conv_body(g_ref, c_ref, w2_ref, b2_ref, w1p_ref, o_ref):
    c = c_ref[...]
    t = (c[:, 0:1] * w1p_ref[0:1, :]
         + c[:, 1:2] * w1p_ref[1:2, :]
         + c[:, 2:3] * w1p_ref[2:3, :])
    tb = jnp.broadcast_to(t[:, None, :], (_BC, _K, _D)).reshape(_BC * _K, _D)
    h1 = jnp.maximum(g_ref[...] - tb, jnp.float32(0.0))
    h2 = jnp.dot(h1.astype(jnp.bfloat16), w2_ref[...].astype(jnp.bfloat16),
                 preferred_element_type=jnp.float32)
    h2 = jnp.maximum(h2 + b2_ref[...], jnp.float32(0.0))
    for i in range(_BC):
        o_ref[i, :] = jnp.max(h2[i * _K:(i + 1) * _K, :], axis=0)


def _conv(gflat, cents, w2, b2r, w1p8):
    return pl.pallas_call(
        _conv_body,
        grid=(_MP // _BC,),
        in_specs=[
            pl.BlockSpec((_BC * _K, _D), lambda i: (i, 0)),
            pl.BlockSpec((_BC, 3), lambda i: (i, 0)),
            pl.BlockSpec((_D, _D), lambda i: (0, 0)),
            pl.BlockSpec((1, _D), lambda i: (0, 0)),
            pl.BlockSpec((8, _D), lambda i: (0, 0)),
        ],
        out_specs=pl.BlockSpec((_BC, _D), lambda i: (i, 0)),
        out_shape=jax.ShapeDtypeStruct((_MP, _D), jnp.float32),
    )(gflat, cents, w2, b2r, w1p8)


def kernel(x, pos, batch, W1, b1, W2, b2):
    pxp = jnp.pad(pos[:, 0], (0, _NP - _N), constant_values=1e9)
    pyp = jnp.pad(pos[:, 1], (0, _NP - _N), constant_values=1e9)
    pzp = jnp.pad(pos[:, 2], (0, _NP - _N), constant_values=1e9)

    posp = jnp.stack([pxp, pyp, pzp], axis=1)
    post = jnp.pad(jnp.transpose(posp), ((0, 5), (0, 0)))
    sel_idx = _fps(pxp.reshape(80, 128), pyp.reshape(80, 128),
                   pzp.reshape(80, 128), post)
    idx = jnp.sort(sel_idx)

    xp = jnp.pad(x, ((0, _NP - _N), (0, 0)))
    w1x = W1[:_D]
    w1p8 = jnp.pad(W1[_D:], ((0, 5), (0, 0)))
    b1r = b1.reshape(1, _D)
    a = _pre(xp, posp, w1x, w1p8, b1r)

    idxp = jnp.pad(idx, (0, _MP - _M))
    cents = posp[idxp]
    cents8 = jnp.pad(cents, ((0, 0), (0, 5)))
    d2 = _d2(cents8, post)

    g = _make_sel()(d2.reshape(_MP * _NP), a)

    out = _conv(g, cents, W2, b2.reshape(1, _D), w1p8)

    new_pos = cents[:_M]
    new_batch = batch[idx]
    return out[:_M], new_pos, new_batch


# speedup vs baseline: 26.2039x; 1.0984x over previous
import functools

import numpy as np

import jax
import jax.numpy as jnp
from jax import lax
from jax.experimental import pallas as pl
from jax.experimental.pallas import tpu as pltpu
from jax.experimental.pallas import tpu_sc as plsc

_N = 10000
_NP = 10240
_M = 2500
_MP = 2560
_D = 128
_K = 128
_CAP = 1024
_R2 = np.float32(0.2 * 0.2)


def _fps_body(px_ref, py_ref, pz_ref, psm_ref, sel_ref, dists_ref):
    fidx = (lax.broadcasted_iota(jnp.int32, (80, 128), 0) * 128
            + lax.broadcasted_iota(jnp.int32, (80, 128), 1))
    valid = fidx < _N
    dists_ref[...] = jnp.where(valid, jnp.inf, -jnp.inf).astype(jnp.float32)

    def body(i, mval):
        dists = dists_ref[...]
        nxt = jnp.min(jnp.where(dists == mval, fidx, jnp.int32(2 ** 30)))
        sel_ref[i] = nxt
        sx = psm_ref[0, nxt]
        sy = psm_ref[1, nxt]
        sz = psm_ref[2, nxt]
        dx = px_ref[...] - sx
        dy = py_ref[...] - sy
        dz = pz_ref[...] - sz
        d = (dx * dx + dy * dy) + dz * dz
        newd = jnp.minimum(dists, d)
        dists_ref[...] = newd
        return jnp.max(newd)

    lax.fori_loop(0, _M, body, jnp.float32(jnp.inf))


def _fps(px2, py2, pz2, post):
    return pl.pallas_call(
        _fps_body,
        in_specs=[
            pl.BlockSpec((80, 128), lambda: (0, 0)),
            pl.BlockSpec((80, 128), lambda: (0, 0)),
            pl.BlockSpec((80, 128), lambda: (0, 0)),
            pl.BlockSpec(memory_space=pltpu.SMEM),
        ],
        out_shape=jax.ShapeDtypeStruct((_M,), jnp.int32),
        out_specs=pl.BlockSpec(memory_space=pltpu.SMEM),
        scratch_shapes=[pltpu.VMEM((80, 128), jnp.float32)],
    )(px2, py2, pz2, post)


def _pre_body(x_ref, p_ref, w1x_ref, w1p_ref, b1_ref, a_ref):
    a = jnp.dot(x_ref[...].astype(jnp.bfloat16),
                w1x_ref[...].astype(jnp.bfloat16),
                preferred_element_type=jnp.float32)
    p = p_ref[...]
    a = a + (p[:, 0:1] * w1p_ref[0:1, :]
             + p[:, 1:2] * w1p_ref[1:2, :]
             + p[:, 2:3] * w1p_ref[2:3, :])
    a_ref[...] = a + b1_ref[...]


def _pre(xp, posp, w1x, w1p8, b1r):
    blk = 1024
    return pl.pallas_call(
        _pre_body,
        grid=(_NP // blk,),
        in_specs=[
            pl.BlockSpec((blk, _D), lambda i: (i, 0)),
            pl.BlockSpec((blk, 3), lambda i: (i, 0)),
            pl.BlockSpec((_D, _D), lambda i: (0, 0)),
            pl.BlockSpec((8, _D), lambda i: (0, 0)),
            pl.BlockSpec((1, _D), lambda i: (0, 0)),
        ],
        out_specs=pl.BlockSpec((blk, _D), lambda i: (i, 0)),
        out_shape=jax.ShapeDtypeStruct((_NP, _D), jnp.float32),
    )(xp, posp, w1x, w1p8, b1r)


_BD = 128


def _d2_body(c8_ref, pt_ref, d2_ref):
    c8 = c8_ref[...]
    c2 = ((c8[:, 0:1] * c8[:, 0:1] + c8[:, 1:2] * c8[:, 1:2])
          + c8[:, 2:3] * c8[:, 2:3])
    pt = pt_ref[...]
    p2 = ((pt[0:1, :] * pt[0:1, :] + pt[1:2, :] * pt[1:2, :])
          + pt[2:3, :] * pt[2:3, :])
    mm = jnp.dot(c8.astype(jnp.bfloat16), pt.astype(jnp.bfloat16),
                 preferred_element_type=jnp.float32)
    d2_ref[...] = (c2 + p2) - 2.0 * mm


def _d2(cents8, post):
    return pl.pallas_call(
        _d2_body,
        grid=(_MP // _BD,),
        in_specs=[
            pl.BlockSpec((_BD, 8), lambda i: (i, 0)),
            pl.BlockSpec((8, _NP), lambda i: (0, 0)),
        ],
        out_specs=pl.BlockSpec((_BD, _NP), lambda i: (i, 0)),
        out_shape=jax.ShapeDtypeStruct((_MP, _NP), jnp.float32),
    )(cents8, post)


@functools.cache
def _make_sel():
    info = plsc.get_sparse_core_info()
    nc, ns = info.num_cores, info.num_subcores
    nw = nc * ns
    rpw = _MP // nw
    mesh = plsc.VectorSubcoreMesh(core_axis_name="c", subcore_axis_name="s")

    @functools.partial(
        pl.kernel,
        out_type=jax.ShapeDtypeStruct((_MP * _K, _D), jnp.float32),
        mesh=mesh,
        compiler_params=pltpu.CompilerParams(needs_layout_passes=False),
        scratch_types=[
            pltpu.VMEM((_NP,), jnp.float32),
            pltpu.VMEM((_NP,), jnp.float32),
            pltpu.VMEM((_CAP,), jnp.float32),
            pltpu.VMEM((_CAP,), jnp.int32),
            pltpu.VMEM((_K + 16,), jnp.int32),
            pltpu.VMEM((_K,), jnp.int32),
            pltpu.VMEM((_K, _D), jnp.float32),
            pltpu.VMEM((_K, _D), jnp.float32),
            pltpu.SemaphoreType.DMA,
            pltpu.SemaphoreType.DMA,
            pltpu.SemaphoreType.DMA,
            pltpu.SemaphoreType.DMA,
            pltpu.SemaphoreType.DMA,
        ],
    )
    def sel(d2_h, a_h, g_h,
            d2v0, d2v1, cdv, civ, nbrv, nbr2, gbuf0, gbuf1,
            semd0, semd1, semg, semw0, semw1):
        wid = lax.axis_index("s") * nc + lax.axis_index("c")
        base = wid * rpw
        iota16 = lax.iota(jnp.int32, 16)
        zz = jnp.zeros((16,), jnp.int32)
        inf16 = jnp.full((16,), jnp.inf, jnp.float32)
        semd = (semd0, semd1)
        semw = (semw0, semw1)
        d2vs = (d2v0, d2v1)
        gbufs = (gbuf0, gbuf1)

        pltpu.make_async_copy(
            d2_h.at[pl.ds(base * _NP, _NP)], d2v0, semd[0]).start()

        def process(row, b):
            d2b = d2vs[b]
            gb = gbufs[b]
            pltpu.make_async_copy(
                d2_h.at[pl.ds(row * _NP, _NP)], d2b, semd[b]).wait()

            @pl.when(row + 1 < jnp.minimum(_M, base + rpw))
            def _():
                pltpu.make_async_copy(
                    d2_h.at[pl.ds((row + 1) * _NP, _NP)],
                    d2vs[1 - b], semd[1 - b]).start()

            @plsc.parallel_loop(0, _NP // 16, unroll=8, carry=jnp.int32(0))
            def cnum(k, cur):
                dv = d2b[pl.ds(k * 16, 16)]
                msk = dv <= _R2
                pc = plsc.all_reduce_population_count(msk)[0]
                cs = jnp.minimum(cur, _CAP - 16)
                plsc.store_compressed(cdv.at[pl.ds(cs, 16)], dv, mask=msk)
                plsc.store_compressed(civ.at[pl.ds(cs, 16)],
                                      k * 16 + iota16, mask=msk)
                return cur + pc
            cdv[pl.ds(jnp.minimum(cnum, _CAP - 16), 16)] = inf16
            nch = (cnum + 15) // 16

            def bisect(_):
                def bis(_, carry2):
                    lo, hi = carry2
                    mid = (lo + hi) * jnp.float32(0.5)
                    mids = jnp.broadcast_to(mid, (16,))

                    @plsc.parallel_loop(0, nch, unroll=4,
                                        carry=jnp.zeros((16,), jnp.int32))
                    def accv(k, acc):
                        dv = cdv[pl.ds(k * 16, 16)]
                        return acc + jnp.where(dv <= mids, 1, 0).astype(
                            jnp.int32)
                    tot = jnp.sum(accv)
                    feas = tot <= _K
                    return (jnp.where(feas, mid, lo),
                            jnp.where(feas, hi, mid))

                lo, _hi = lax.fori_loop(0, 36, bis,
                                        (jnp.float32(-1.0), _R2))
                return lo

            tfin = lax.cond(cnum > _K, bisect,
                            lambda _: jnp.float32(1.0), 0)
            tfs = jnp.broadcast_to(tfin, (16,))

            @plsc.parallel_loop(0, nch, unroll=4, carry=jnp.int32(0))
            def c2n(k, cur):
                dv = cdv[pl.ds(k * 16, 16)]
                iv = civ[pl.ds(k * 16, 16)]
                msk = dv <= tfs
                pc = plsc.all_reduce_population_count(msk)[0]
                cs = jnp.minimum(cur, _K)
                plsc.store_compressed(nbrv.at[pl.ds(cs, 16)], iv, mask=msk)
                return cur + pc
            n0 = plsc.load_gather(nbrv, [zz])
            c2s = jnp.broadcast_to(c2n, (16,))

            def pad(k, carry2):
                cu = nbrv[pl.ds(k * 16, 16)]
                pv = k * 16 + iota16
                nbr2[pl.ds(k * 16, 16)] = jnp.where(pv < c2s, cu, n0)
                return carry2

            lax.fori_loop(0, _K // 16, pad, 0)

            pltpu.async_copy(a_h.at[nbr2], gb, semg).wait()
            pltpu.make_async_copy(
                gb, g_h.at[pl.ds(row * _K, _K)], semw[b]).start()

        def pair(r2, carry):
            for b in (0, 1):
                row = base + r2 * 2 + b

                @pl.when(row < _M)
                def _():
                    @pl.when(r2 > 0)
                    def _():
                        pltpu.make_async_copy(
                            gbufs[b], g_h.at[pl.ds(row * _K, _K)],
                            semw[b]).wait()

                    process(row, b)

            return carry

        lax.fori_loop(0, rpw // 2, pair, 0)
        pltpu.make_async_copy(
            gbuf0, g_h.at[pl.ds(base * _K, _K)], semw[0]).wait()
        pltpu.make_async_copy(
            gbuf1, g_h.at[pl.ds(base * _K, _K)], semw[1]).wait()

    return sel


_BC = 16


def _conv_body(g_ref, c_ref, w2_ref, b2_ref, w1p_ref, o_ref):
    c = c_ref[...]
    t = (c[:, 0:1] * w1p_ref[0:1, :]
         + c[:, 1:2] * w1p_ref[1:2, :]
         + c[:, 2:3] * w1p_ref[2:3, :])
    tb = jnp.broadcast_to(t[:, None, :], (_BC, _K, _D)).reshape(_BC * _K, _D)
    h1 = jnp.maximum(g_ref[...] - tb, jnp.float32(0.0))
    h2 = jnp.dot(h1.astype(jnp.bfloat16), w2_ref[...].astype(jnp.bfloat16),
                 preferred_element_type=jnp.float32)
    h2 = jnp.maximum(h2 + b2_ref[...], jnp.float32(0.0))
    for i in range(_BC):
        o_ref[i, :] = jnp.max(h2[i * _K:(i + 1) * _K, :], axis=0)


def _conv(gflat, cents, w2, b2r, w1p8):
    return pl.pallas_call(
        _conv_body,
        grid=(_MP // _BC,),
        in_specs=[
            pl.BlockSpec((_BC * _K, _D), lambda i: (i, 0)),
            pl.BlockSpec((_BC, 3), lambda i: (i, 0)),
            pl.BlockSpec((_D, _D), lambda i: (0, 0)),
            pl.BlockSpec((1, _D), lambda i: (0, 0)),
            pl.BlockSpec((8, _D), lambda i: (0, 0)),
        ],
        out_specs=pl.BlockSpec((_BC, _D), lambda i: (i, 0)),
        out_shape=jax.ShapeDtypeStruct((_MP, _D), jnp.float32),
    )(gflat, cents, w2, b2r, w1p8)


def kernel(x, pos, batch, W1, b1, W2, b2):
    pxp = jnp.pad(pos[:, 0], (0, _NP - _N), constant_values=1e9)
    pyp = jnp.pad(pos[:, 1], (0, _NP - _N), constant_values=1e9)
    pzp = jnp.pad(pos[:, 2], (0, _NP - _N), constant_values=1e9)

    posp = jnp.stack([pxp, pyp, pzp], axis=1)
    post = jnp.pad(jnp.transpose(posp), ((0, 5), (0, 0)))
    sel_idx = _fps(pxp.reshape(80, 128), pyp.reshape(80, 128),
                   pzp.reshape(80, 128), post)
    idx = jnp.sort(sel_idx)

    xp = jnp.pad(x, ((0, _NP - _N), (0, 0)))
    w1x = W1[:_D]
    w1p8 = jnp.pad(W1[_D:], ((0, 5), (0, 0)))
    b1r = b1.reshape(1, _D)
    a = _pre(xp, posp, w1x, w1p8, b1r)

    idxp = jnp.pad(idx, (0, _MP - _M))
    cents = posp[idxp]
    cents8 = jnp.pad(cents, ((0, 0), (0, 5)))
    d2 = _d2(cents8, post)

    g = _make_sel()(d2.reshape(_MP * _NP), a)

    out = _conv(g, cents, W2, b2.reshape(1, _D), w1p8)

    new_pos = cents[:_M]
    new_batch = batch[idx]
    return out[:_M], new_pos, new_batch
